# baseline (speedup 1.0000x reference)
import jax
import jax.numpy as jnp
from jax import lax
from jax.experimental import pallas as pl
from jax.experimental.pallas import tpu as pltpu
from jax.experimental.pallas import tpu_sc as plsc

D_VOCAB = 100000
N_ROWS = 1024
NC = 2
NS = 16
L = 16
NW = NC * NS
ROWS_PER_W = N_ROWS // NW


def _body(x_hbm, out_hbm, idx_v, rowbuf):
    wid = lax.axis_index("s") * NC + lax.axis_index("c")
    base = wid * ROWS_PER_W

    pltpu.sync_copy(x_hbm.at[pl.ds(base * 2, 2 * ROWS_PER_W)], idx_v)

    zeros16 = jnp.zeros((L,), jnp.float32)
    ones16 = jnp.ones((L,), jnp.float32)
    iota16 = lax.iota(jnp.int32, L)

    def _zero(i, carry):
        rowbuf[pl.ds(i * L, L)] = zeros16
        return carry

    lax.fori_loop(0, D_VOCAB // L, _zero, 0)

    for c in range(ROWS_PER_W // 8):
        chunk = idx_v[pl.ds(c * L, L)]
        for k in range(8):
            r = c * 8 + k
            m0 = iota16 == (2 * k)
            m1 = iota16 == (2 * k + 1)
            plsc.addupdate_scatter(rowbuf, [chunk], ones16, mask=m0)
            plsc.addupdate_scatter(rowbuf, [chunk], ones16, mask=m1)
            pltpu.sync_copy(rowbuf, out_hbm.at[base + r])
            plsc.store_scatter(rowbuf, [chunk], zeros16, mask=m0 | m1)


@jax.jit
def _embed(x_flat):
    mesh = plsc.VectorSubcoreMesh(
        core_axis_name="c", subcore_axis_name="s", num_cores=NC,
        num_subcores=NS)
    f = pl.kernel(
        _body,
        out_type=jax.ShapeDtypeStruct((N_ROWS, D_VOCAB), jnp.float32),
        mesh=mesh,
        scratch_types=[
            pltpu.VMEM((2 * ROWS_PER_W,), jnp.int32),
            pltpu.VMEM((D_VOCAB,), jnp.float32),
        ],
        compiler_params=pltpu.CompilerParams(needs_layout_passes=False),
    )
    return f(x_flat)


def kernel(x, W_E):
    del W_E
    out = _embed(x.reshape(-1).astype(jnp.int32))
    return out[:, None, :]
